# baseline (speedup 1.0000x reference)
import functools

import jax
import jax.numpy as jnp
from jax import lax
from jax.experimental import pallas as pl
from jax.experimental.pallas import tpu as pltpu
from jax.experimental.pallas import tpu_sc as plsc

_N = 32768
_NSLICE = _N // 16
_BISECT_ITERS = 13
_SECANT_ITERS = 2
_SC_ROWS = 32


def _tc_block(z_ref, u_ref, out_ref):
    z = z_ref[...]
    u = u_ref[...]
    lo = jnp.min(z - u, axis=-1, keepdims=True) - 1.0
    hi = jnp.max(z, axis=-1, keepdims=True)

    def eval_f(tau):
        return jnp.sum(jnp.clip(z - tau, 0.0, u), axis=-1, keepdims=True) - 1.0

    tau_p = jnp.zeros_like(lo)
    f_p = jnp.zeros_like(lo)
    tau_c = jnp.zeros_like(lo)
    f_c = jnp.zeros_like(lo)
    for _ in range(_BISECT_ITERS):
        mid = 0.5 * (lo + hi)
        fm = eval_f(mid)
        pos = fm > 0.0
        lo = jnp.where(pos, mid, lo)
        hi = jnp.where(pos, hi, mid)
        tau_p, f_p = tau_c, f_c
        tau_c, f_c = mid, fm

    for _ in range(_SECANT_ITERS):
        denom = f_c - f_p
        ok = denom != 0.0
        tau_s = tau_c - f_c * (tau_c - tau_p) / jnp.where(ok, denom, 1.0)
        good = ok & (tau_s > lo) & (tau_s < hi)
        tau_n = jnp.where(good, tau_s, 0.5 * (lo + hi))
        fn = eval_f(tau_n)
        pos = fn > 0.0
        lo = jnp.where(pos, tau_n, lo)
        hi = jnp.where(pos, hi, tau_n)
        tau_p, f_p = tau_c, f_c
        tau_c, f_c = tau_n, fn

    t = z - tau_c
    free = (t > 0.0) & (t < u)
    nA = jnp.sum(free.astype(z.dtype), axis=-1, keepdims=True)
    C = jnp.sum(jnp.clip(t, 0.0, u), axis=-1, keepdims=True)
    f = C - 1.0
    pos = f > 0.0
    lo = jnp.where(pos, tau_c, lo)
    hi = jnp.where(pos, hi, tau_c)
    tau_s = tau_c + f / jnp.maximum(nA, 1.0)
    good = (nA > 0.0) & (((tau_s > lo) & (tau_s < hi)) | (tau_s == tau_c))
    tau = jnp.where(good, tau_s, 0.5 * (lo + hi))

    t = z - tau
    ltu = t < u
    free = (t > 0.0) & ltu
    nA = jnp.sum(free.astype(z.dtype), axis=-1, keepdims=True)
    sAB = jnp.sum(jnp.where(free, z, jnp.where(ltu, 0.0, u)),
                  axis=-1, keepdims=True)
    tau_f = (sAB - 1.0) / jnp.maximum(nA, 1.0)
    tau_f = jnp.where(nA > 0.0, tau_f, tau)
    out_ref[...] = jnp.where(free, z - tau_f, jnp.where(ltu, 0.0, u))


def _tc_solve(z, u, block_rows=32):
    rows, n = z.shape
    grid = (rows // block_rows,)
    spec = pl.BlockSpec((block_rows, n), lambda i: (i, 0))
    return pl.pallas_call(
        _tc_block,
        grid=grid,
        in_specs=[spec, spec],
        out_specs=spec,
        out_shape=jax.ShapeDtypeStruct(z.shape, z.dtype),
    )(z, u)


_GATHER_DNUMS = lax.GatherDimensionNumbers(
    offset_dims=(), collapsed_slice_dims=(0,), start_index_map=(0,))


def _take16(x, idx):
    return lax.gather(x, idx[:, None], _GATHER_DNUMS, slice_sizes=(1,),
                      mode=lax.GatherScatterMode.PROMISE_IN_BOUNDS)


def _xl_reduce(x, op):
    idx = lax.iota(jnp.int32, 16)
    for k in (8, 4, 2, 1):
        x = op(x, _take16(x, idx ^ k))
    return x


def _sc_row_solver(zb, ub, ob):
    def slice_red(body_fn, init):
        def step(i, accs):
            out = []
            for j in range(4):
                off = (i * 4 + j) * 16
                zs = zb[pl.ds(off, 16)]
                us = ub[pl.ds(off, 16)]
                out.append(body_fn(accs[j], zs, us))
            return tuple(out)
        accs = lax.fori_loop(0, _NSLICE // 4, step, (init,) * 4, unroll=2)
        a, b, c, d = accs
        return jax.tree.map(lambda p, q, r, s: (p + q) + (r + s), a, b, c, d)

    big = jnp.float32(3.0e38)
    lo16, hi16 = slice_red(
        lambda acc, zs, us: (jnp.minimum(acc[0], zs - us),
                             jnp.maximum(acc[1], zs)),
        (jnp.full((16,), big), jnp.full((16,), -big)),
    )
    lo = -_xl_reduce(-lo16, jnp.maximum) - 1.0
    hi = _xl_reduce(hi16, jnp.maximum)

    def eval_f(tau):
        acc = slice_red(
            lambda acc, zs, us: acc + jnp.clip(zs - tau, 0.0, us),
            jnp.zeros((16,), jnp.float32),
        )
        return _xl_reduce(acc, jnp.add) - 1.0

    tau_p = jnp.zeros((16,), jnp.float32)
    f_p = jnp.zeros((16,), jnp.float32)
    tau_c = jnp.zeros((16,), jnp.float32)
    f_c = jnp.zeros((16,), jnp.float32)
    for _ in range(_BISECT_ITERS):
        mid = 0.5 * (lo + hi)
        fm = eval_f(mid)
        pos = fm > 0.0
        lo = jnp.where(pos, mid, lo)
        hi = jnp.where(pos, hi, mid)
        tau_p, f_p = tau_c, f_c
        tau_c, f_c = mid, fm

    for _ in range(_SECANT_ITERS):
        denom = f_c - f_p
        ok = denom != 0.0
        tau_s = tau_c - f_c * (tau_c - tau_p) / jnp.where(ok, denom, 1.0)
        good = ok & (tau_s > lo) & (tau_s < hi)
        tau_n = jnp.where(good, tau_s, 0.5 * (lo + hi))
        fn = eval_f(tau_n)
        pos = fn > 0.0
        lo = jnp.where(pos, tau_n, lo)
        hi = jnp.where(pos, hi, tau_n)
        tau_p, f_p = tau_c, f_c
        tau_c, f_c = tau_n, fn

    def cn_body(acc, zs, us):
        t = zs - tau_c
        c = jnp.clip(t, 0.0, us)
        freef = jnp.where((t > 0.0) & (t < us), 1.0, 0.0)
        return (acc[0] + c, acc[1] + freef)

    C16, nA16 = slice_red(cn_body, (jnp.zeros((16,), jnp.float32),
                                    jnp.zeros((16,), jnp.float32)))
    C = _xl_reduce(C16, jnp.add)
    nA = _xl_reduce(nA16, jnp.add)
    f = C - 1.0
    pos = f > 0.0
    lo = jnp.where(pos, tau_c, lo)
    hi = jnp.where(pos, hi, tau_c)
    tau_s = tau_c + f / jnp.maximum(nA, 1.0)
    good = (nA > 0.0) & (((tau_s > lo) & (tau_s < hi)) | (tau_s == tau_c))
    tau = jnp.where(good, tau_s, 0.5 * (lo + hi))

    def sn_body(acc, zs, us):
        t = zs - tau
        ltu = t < us
        freef = jnp.where((t > 0.0) & ltu, 1.0, 0.0)
        sab = jnp.where((t > 0.0) & ltu, zs, jnp.where(ltu, 0.0, us))
        return (acc[0] + sab, acc[1] + freef)

    s16, n16 = slice_red(sn_body, (jnp.zeros((16,), jnp.float32),
                                   jnp.zeros((16,), jnp.float32)))
    sAB = _xl_reduce(s16, jnp.add)
    nAf = _xl_reduce(n16, jnp.add)
    tau_f = (sAB - 1.0) / jnp.maximum(nAf, 1.0)
    tau_f = jnp.where(nAf > 0.0, tau_f, tau)

    def out_step(i, carry):
        zs = zb[pl.ds(i * 16, 16)]
        us = ub[pl.ds(i * 16, 16)]
        t = zs - tau
        ltu = t < us
        free = (t > 0.0) & ltu
        ob[pl.ds(i * 16, 16)] = jnp.where(free, zs - tau_f,
                                          jnp.where(ltu, 0.0, us))
        return carry

    lax.fori_loop(0, _NSLICE, out_step, 0, unroll=8)


def _sc_body(z_ref, u_ref, o_ref, zb, ub, ob, sem_in, sem_out):
    c = lax.axis_index("c")
    s = lax.axis_index("s")
    row = c * 16 + s
    cp_z = pltpu.make_async_copy(z_ref.at[row], zb, sem_in)
    cp_z.start()
    cp_u = pltpu.make_async_copy(u_ref.at[row], ub, sem_in)
    cp_u.start()
    cp_z.wait()
    cp_u.wait()
    _sc_row_solver(zb, ub, ob)
    cp_o = pltpu.make_async_copy(ob, o_ref.at[row], sem_out)
    cp_o.start()
    cp_o.wait()


def _sc_solve(z, u):
    rows = z.shape[0]
    mesh = plsc.VectorSubcoreMesh(core_axis_name="c", subcore_axis_name="s")
    f = pl.kernel(
        _sc_body,
        out_type=jax.ShapeDtypeStruct((rows, _N), jnp.float32),
        mesh=mesh,
        scratch_types=[
            pltpu.VMEM((_N,), jnp.float32),
            pltpu.VMEM((_N,), jnp.float32),
            pltpu.VMEM((_N,), jnp.float32),
            pltpu.SemaphoreType.DMA,
            pltpu.SemaphoreType.DMA,
        ],
        cost_estimate=pl.CostEstimate(
            flops=90_000_000, bytes_accessed=13_000_000, transcendentals=0),
    )
    return f(z, u)


@jax.jit
def _hybrid(z, u):
    p_sc = _sc_solve(z[-_SC_ROWS:], u[-_SC_ROWS:])
    p_tc = _tc_solve(z[:-_SC_ROWS], u[:-_SC_ROWS])
    return jnp.concatenate([p_tc, p_sc], axis=0)


def kernel(input1, input2):
    return _hybrid(input1, input2)


# speedup vs baseline: 1.6829x; 1.6829x over previous
import functools

import jax
import jax.numpy as jnp
from jax.experimental import pallas as pl

_BISECT_ITERS = 13
_SECANT_ITERS = 2


def _csparsemax_block(z_ref, u_ref, out_ref):
    z = z_ref[...]
    u = u_ref[...]
    lo = jnp.min(z - u, axis=-1, keepdims=True) - 1.0
    hi = jnp.max(z, axis=-1, keepdims=True)

    def eval_f(tau):
        return jnp.sum(jnp.clip(z - tau, 0.0, u), axis=-1, keepdims=True) - 1.0

    tau_p = jnp.zeros_like(lo)
    f_p = jnp.zeros_like(lo)
    tau_c = jnp.zeros_like(lo)
    f_c = jnp.zeros_like(lo)
    for _ in range(_BISECT_ITERS):
        mid = 0.5 * (lo + hi)
        fm = eval_f(mid)
        pos = fm > 0.0
        lo = jnp.where(pos, mid, lo)
        hi = jnp.where(pos, hi, mid)
        tau_p, f_p = tau_c, f_c
        tau_c, f_c = mid, fm

    for _ in range(_SECANT_ITERS):
        denom = f_c - f_p
        ok = denom != 0.0
        tau_s = tau_c - f_c * (tau_c - tau_p) / jnp.where(ok, denom, 1.0)
        good = ok & (tau_s > lo) & (tau_s < hi)
        tau_n = jnp.where(good, tau_s, 0.5 * (lo + hi))
        fn = eval_f(tau_n)
        pos = fn > 0.0
        lo = jnp.where(pos, tau_n, lo)
        hi = jnp.where(pos, hi, tau_n)
        tau_p, f_p = tau_c, f_c
        tau_c, f_c = tau_n, fn

    t = z - tau_c
    free = (t > 0.0) & (t < u)
    nA = jnp.sum(free.astype(z.dtype), axis=-1, keepdims=True)
    C = jnp.sum(jnp.clip(t, 0.0, u), axis=-1, keepdims=True)
    f = C - 1.0
    pos = f > 0.0
    lo = jnp.where(pos, tau_c, lo)
    hi = jnp.where(pos, hi, tau_c)
    tau_s = tau_c + f / jnp.maximum(nA, 1.0)
    good = (nA > 0.0) & (((tau_s > lo) & (tau_s < hi)) | (tau_s == tau_c))
    tau = jnp.where(good, tau_s, 0.5 * (lo + hi))

    t = z - tau
    ltu = t < u
    free = (t > 0.0) & ltu
    nA = jnp.sum(free.astype(z.dtype), axis=-1, keepdims=True)
    sAB = jnp.sum(jnp.where(free, z, jnp.where(ltu, 0.0, u)),
                  axis=-1, keepdims=True)
    tau_f = (sAB - 1.0) / jnp.maximum(nA, 1.0)
    tau_f = jnp.where(nA > 0.0, tau_f, tau)
    out_ref[...] = jnp.where(free, z - tau_f, jnp.where(ltu, 0.0, u))


@functools.partial(jax.jit, static_argnames=("block_rows",))
def _csparsemax(z, u, block_rows=32):
    rows, n = z.shape
    grid = (rows // block_rows,)
    spec = pl.BlockSpec((block_rows, n), lambda i: (i, 0))
    return pl.pallas_call(
        _csparsemax_block,
        grid=grid,
        in_specs=[spec, spec],
        out_specs=spec,
        out_shape=jax.ShapeDtypeStruct(z.shape, z.dtype),
    )(z, u)


def kernel(input1, input2):
    return _csparsemax(input1, input2)
